# baseline (speedup 1.0000x reference)
import functools

import jax
import jax.numpy as jnp
import numpy as np
from jax import lax
from jax.experimental import pallas as pl
from jax.experimental.pallas import tpu as pltpu
from jax.experimental.pallas import tpu_sc as plsc

_VOCAB = 1000
_D = 128
_DW = _D // 2
_POS = 4
_NC = 2
_NS = 16
_NW = _NC * _NS
_LANES = 16

def _table_body(embed_ref, w_ref, b_ref, tbl_ref):
    wp = jnp.concatenate(
        [w_ref[pl.ds(32 * m + h, 16), :] for h in (0, 16) for m in range(4)],
        axis=0)
    bp = jnp.concatenate(
        [b_ref[pl.ds(32 * m + h, 16)] for h in (0, 16) for m in range(4)])
    bias = bp * (1.0 / _POS)
    for j in range(_POS):
        wp_j = wp[:, j * _D:(j + 1) * _D]
        t = (
            lax.dot_general(
                embed_ref[...],
                wp_j,
                dimension_numbers=(((1,), (1,)), ((), ())),
                preferred_element_type=jnp.float32,
            )
            + bias[None, :]
        )
        t16 = lax.bitcast_convert_type(
            t.astype(jnp.bfloat16), jnp.uint16).astype(jnp.uint32)
        tbl_ref[pl.ds(j * _VOCAB, _VOCAB), :] = lax.bitcast_convert_type(
            t16[:, :_DW] | (t16[:, _DW:] << 16), jnp.int32)


def _make_fused_table(embed, W, b):
    return pl.pallas_call(
        _table_body,
        out_shape=jax.ShapeDtypeStruct((_POS * _VOCAB, _DW), jnp.int32),
    )(embed, W, b)


def _sc_body(tok_ref, tbl_ref, out_ref,
             tok_v, idx_v, rows0, rows1, outc0, outc1,
             sem_g0, sem_g1, sem_o0, sem_o1,
             *, batch, b_per_w, cb):
    wid = lax.axis_index("s") * _NC + lax.axis_index("c")
    base = wid * b_per_w
    n_chunks = b_per_w // cb
    g = cb * _POS
    rows_bufs = (rows0, rows1)
    outc_bufs = (outc0, outc1)
    sem_g = (sem_g0, sem_g1)
    sem_o = (sem_o0, sem_o1)
    sh16 = jnp.full((_LANES,), 16, jnp.int32)

    for j in range(_POS):
        pltpu.async_copy(tok_ref.at[pl.ds(j * batch + base, b_per_w)],
                         tok_v.at[pl.ds(j * b_per_w, b_per_w)], sem_o0)
    for j in range(_POS):
        pltpu.make_async_copy(tok_ref.at[pl.ds(0, b_per_w)],
                              tok_v.at[pl.ds(0, b_per_w)], sem_o0).wait()

    def build_idx(c):
        for j in range(_POS):
            for s in range(cb // _LANES):
                src = pl.ds(j * b_per_w + c * cb + s * _LANES, _LANES)
                idx_v[c, j, pl.ds(s * _LANES, _LANES)] = (
                    tok_v[src] + (j * _VOCAB))

    def fire(c, rows_b, sem):
        for j in range(_POS):
            pltpu.async_copy(tbl_ref.at[idx_v.at[c, j]],
                             rows_b.at[pl.ds(j * cb, cb)], sem)

    build_idx(0)
    fire(0, rows0, sem_g0)
    build_idx(1)
    fire(1, rows1, sem_g1)
    for c in range(2, n_chunks):
        build_idx(c)

    def pair_body(p, _):
        for bsel in range(2):
            c = p * 2 + bsel
            rows_b = rows_bufs[bsel]
            outc_b = outc_bufs[bsel]
            for j in range(_POS):
                pltpu.make_async_copy(tbl_ref.at[idx_v.at[0, 0]],
                                      rows_b.at[pl.ds(j * cb, cb)],
                                      sem_g[bsel]).wait()
            @pl.when(c >= 2)
            def _():
                pltpu.make_async_copy(outc_b, out_ref.at[pl.ds(0, cb)],
                                      sem_o[bsel]).wait()

            @plsc.parallel_loop(0, cb, 1, unroll=4)
            def _(r):
                for m in range(_DW // _LANES):
                    sl = pl.ds(m * _LANES, _LANES)
                    v = rows_b[r, sl]
                    acc_lo = plsc.bitcast(lax.shift_left(v, sh16), jnp.float32)
                    acc_hi = plsc.bitcast(v, jnp.float32)
                    for j in range(1, _POS):
                        v = rows_b[j * cb + r, sl]
                        acc_lo = acc_lo + plsc.bitcast(
                            lax.shift_left(v, sh16), jnp.float32)
                        acc_hi = acc_hi + plsc.bitcast(v, jnp.float32)
                    outc_b[r, pl.ds(m * 32, _LANES)] = acc_lo
                    outc_b[r, pl.ds(m * 32 + _LANES, _LANES)] = acc_hi

            row0 = base + c * cb
            pltpu.async_copy(outc_b, out_ref.at[pl.ds(row0, cb)],
                             sem_o[bsel])

            @pl.when(c + 2 < n_chunks)
            def _():
                fire(c + 2, rows_b, sem_g[bsel])
        return 0

    lax.fori_loop(0, n_chunks // 2, pair_body, 0)

    for bsel in range(2):
        pltpu.make_async_copy(outc_bufs[bsel], out_ref.at[pl.ds(0, cb)],
                              sem_o[bsel]).wait()


def _gather_sum(tokens_pm, tbl, batch):
    b_per_w = batch // _NW
    cb = 128
    n_chunks = b_per_w // cb
    mesh = plsc.VectorSubcoreMesh(
        core_axis_name="c", subcore_axis_name="s",
        num_cores=_NC, num_subcores=_NS,
    )
    g = cb * _POS
    run = pl.kernel(
        functools.partial(_sc_body, batch=batch, b_per_w=b_per_w, cb=cb),
        out_type=jax.ShapeDtypeStruct((batch, _D), jnp.float32),
        mesh=mesh,
        compiler_params=pltpu.CompilerParams(
            needs_layout_passes=False,
            use_tc_tiling_on_sc=False,
        ),
        scratch_types=[
            pltpu.VMEM((_POS * b_per_w,), jnp.int32),
            pltpu.VMEM((n_chunks, _POS, cb), jnp.int32),
            pltpu.VMEM((g, _DW), jnp.int32),
            pltpu.VMEM((g, _DW), jnp.int32),
            pltpu.VMEM((cb, _D), jnp.float32),
            pltpu.VMEM((cb, _D), jnp.float32),
            pltpu.SemaphoreType.DMA,
            pltpu.SemaphoreType.DMA,
            pltpu.SemaphoreType.DMA,
            pltpu.SemaphoreType.DMA,
        ],
    )
    return run(tokens_pm, tbl)


def kernel(ctrl_tokens, embed, W, b):
    batch = ctrl_tokens.shape[0]
    tokens_pm = ctrl_tokens.astype(jnp.int32).T.reshape(-1)
    tbl = _make_fused_table(embed, W, b)
    out = _gather_sum(tokens_pm, tbl, batch)
    return out[..., None]


# speedup vs baseline: 1.0164x; 1.0164x over previous
import functools

import jax
import jax.numpy as jnp
import numpy as np
from jax import lax
from jax.experimental import pallas as pl
from jax.experimental.pallas import tpu as pltpu
from jax.experimental.pallas import tpu_sc as plsc

_VOCAB = 1000
_D = 128
_DW = _D // 2
_POS = 4
_NC = 2
_NS = 16
_NW = _NC * _NS
_LANES = 16

def _table_body(embed_ref, w_ref, b_ref, tbl_ref):
    wp = jnp.concatenate(
        [w_ref[pl.ds(32 * m + h, 16), :] for h in (0, 16) for m in range(4)],
        axis=0)
    bp = jnp.concatenate(
        [b_ref[pl.ds(32 * m + h, 16)] for h in (0, 16) for m in range(4)])
    bias = bp * (1.0 / _POS)
    for j in range(_POS):
        wp_j = wp[:, j * _D:(j + 1) * _D]
        t = (
            lax.dot_general(
                embed_ref[...],
                wp_j,
                dimension_numbers=(((1,), (1,)), ((), ())),
                preferred_element_type=jnp.float32,
            )
            + bias[None, :]
        )
        t16 = lax.bitcast_convert_type(
            t.astype(jnp.bfloat16), jnp.uint16).astype(jnp.uint32)
        tbl_ref[pl.ds(j * _VOCAB, _VOCAB), :] = lax.bitcast_convert_type(
            t16[:, :_DW] | (t16[:, _DW:] << 16), jnp.int32)


def _make_fused_table(embed, W, b):
    return pl.pallas_call(
        _table_body,
        out_shape=jax.ShapeDtypeStruct((_POS * _VOCAB, _DW), jnp.int32),
    )(embed, W, b)


def _sc_body(tok_ref, tbl_ref, out_ref,
             tok_v, idx_v, rows0, rows1, outc0, outc1,
             sem_g0, sem_g1, sem_o0, sem_o1,
             *, batch, b_per_w, cb):
    wid = lax.axis_index("s") * _NC + lax.axis_index("c")
    base = wid * b_per_w
    n_chunks = b_per_w // cb
    g = cb * _POS
    rows_bufs = (rows0, rows1)
    outc_bufs = (outc0, outc1)
    sem_g = (sem_g0, sem_g1)
    sem_o = (sem_o0, sem_o1)
    sh16 = jnp.full((_LANES,), 16, jnp.int32)

    for j in range(_POS):
        pltpu.async_copy(tok_ref.at[pl.ds(j * batch + base, b_per_w)],
                         tok_v.at[pl.ds(j * b_per_w, b_per_w)], sem_o0)
    for j in range(_POS):
        pltpu.make_async_copy(tok_ref.at[pl.ds(0, b_per_w)],
                              tok_v.at[pl.ds(0, b_per_w)], sem_o0).wait()

    def build_idx(c):
        for j in range(_POS):
            for s in range(cb // _LANES):
                src = pl.ds(j * b_per_w + c * cb + s * _LANES, _LANES)
                idx_v[c, j // 2, pl.ds((j % 2) * cb + s * _LANES, _LANES)] = (
                    tok_v[src] + (j * _VOCAB))

    half = cb * _POS // 2

    def fire(c, rows_b, sem):
        pltpu.async_copy(tbl_ref.at[idx_v.at[c, 0]],
                         rows_b.at[pl.ds(0, half)], sem)
        pltpu.async_copy(tbl_ref.at[idx_v.at[c, 1]],
                         rows_b.at[pl.ds(half, half)], sem)

    build_idx(0)
    fire(0, rows0, sem_g0)
    build_idx(1)
    fire(1, rows1, sem_g1)

    @plsc.parallel_loop(2, n_chunks, 1, unroll=2)
    def _(c):
        for j in range(_POS):
            for s in range(cb // _LANES):
                src = pl.ds(j * b_per_w + c * cb + s * _LANES, _LANES)
                idx_v[c, j // 2, pl.ds((j % 2) * cb + s * _LANES, _LANES)] = (
                    tok_v[src] + (j * _VOCAB))

    def pair_body(p, _):
        for bsel in range(2):
            c = p * 2 + bsel
            rows_b = rows_bufs[bsel]
            outc_b = outc_bufs[bsel]
            for h in range(2):
                pltpu.make_async_copy(tbl_ref.at[idx_v.at[0, 0]],
                                      rows_b.at[pl.ds(h * half, half)],
                                      sem_g[bsel]).wait()
            @pl.when(c >= 2)
            def _():
                pltpu.make_async_copy(outc_b, out_ref.at[pl.ds(0, cb)],
                                      sem_o[bsel]).wait()

            @plsc.parallel_loop(0, cb, 1, unroll=8)
            def _(r):
                for m in range(_DW // _LANES):
                    sl = pl.ds(m * _LANES, _LANES)
                    v = rows_b[r, sl]
                    acc_lo = plsc.bitcast(lax.shift_left(v, sh16), jnp.float32)
                    acc_hi = plsc.bitcast(v, jnp.float32)
                    for j in range(1, _POS):
                        v = rows_b[j * cb + r, sl]
                        acc_lo = acc_lo + plsc.bitcast(
                            lax.shift_left(v, sh16), jnp.float32)
                        acc_hi = acc_hi + plsc.bitcast(v, jnp.float32)
                    outc_b[r, pl.ds(m * 32, _LANES)] = acc_lo
                    outc_b[r, pl.ds(m * 32 + _LANES, _LANES)] = acc_hi

            row0 = base + c * cb
            pltpu.async_copy(outc_b, out_ref.at[pl.ds(row0, cb)],
                             sem_o[bsel])

            @pl.when(c + 2 < n_chunks)
            def _():
                fire(c + 2, rows_b, sem_g[bsel])
        return 0

    lax.fori_loop(0, n_chunks // 2, pair_body, 0)

    for bsel in range(2):
        pltpu.make_async_copy(outc_bufs[bsel], out_ref.at[pl.ds(0, cb)],
                              sem_o[bsel]).wait()


def _gather_sum(tokens_pm, tbl, batch):
    b_per_w = batch // _NW
    cb = 64
    n_chunks = b_per_w // cb
    mesh = plsc.VectorSubcoreMesh(
        core_axis_name="c", subcore_axis_name="s",
        num_cores=_NC, num_subcores=_NS,
    )
    g = cb * _POS
    run = pl.kernel(
        functools.partial(_sc_body, batch=batch, b_per_w=b_per_w, cb=cb),
        out_type=jax.ShapeDtypeStruct((batch, _D), jnp.float32),
        mesh=mesh,
        compiler_params=pltpu.CompilerParams(
            needs_layout_passes=False,
            use_tc_tiling_on_sc=False,
        ),
        scratch_types=[
            pltpu.VMEM((_POS * b_per_w,), jnp.int32),
            pltpu.VMEM((n_chunks, 2, g // 2), jnp.int32),
            pltpu.VMEM((g, _DW), jnp.int32),
            pltpu.VMEM((g, _DW), jnp.int32),
            pltpu.VMEM((cb, _D), jnp.float32),
            pltpu.VMEM((cb, _D), jnp.float32),
            pltpu.SemaphoreType.DMA,
            pltpu.SemaphoreType.DMA,
            pltpu.SemaphoreType.DMA,
            pltpu.SemaphoreType.DMA,
        ],
    )
    return run(tokens_pm, tbl)


def kernel(ctrl_tokens, embed, W, b):
    batch = ctrl_tokens.shape[0]
    tokens_pm = ctrl_tokens.astype(jnp.int32).T.reshape(-1)
    tbl = _make_fused_table(embed, W, b)
    out = _gather_sum(tokens_pm, tbl, batch)
    return out[..., None]


# speedup vs baseline: 1.0234x; 1.0069x over previous
import functools

import jax
import jax.numpy as jnp
import numpy as np
from jax import lax
from jax.experimental import pallas as pl
from jax.experimental.pallas import tpu as pltpu
from jax.experimental.pallas import tpu_sc as plsc

_VOCAB = 1000
_D = 128
_DW = _D // 2
_POS = 4
_NC = 2
_NS = 16
_NW = _NC * _NS
_LANES = 16

def _table_body(embed_ref, w_ref, b_ref, tbl_ref):
    wp = jnp.concatenate(
        [w_ref[pl.ds(32 * m + h, 16), :] for h in (0, 16) for m in range(4)],
        axis=0)
    bp = jnp.concatenate(
        [b_ref[pl.ds(32 * m + h, 16)] for h in (0, 16) for m in range(4)])
    bias = bp * (1.0 / _POS)
    for j in range(_POS):
        wp_j = wp[:, j * _D:(j + 1) * _D]
        t = (
            lax.dot_general(
                embed_ref[...],
                wp_j,
                dimension_numbers=(((1,), (1,)), ((), ())),
                preferred_element_type=jnp.float32,
            )
            + bias[None, :]
        )
        t16 = lax.bitcast_convert_type(
            t.astype(jnp.bfloat16), jnp.uint16).astype(jnp.uint32)
        tbl_ref[pl.ds(j * _VOCAB, _VOCAB), :] = lax.bitcast_convert_type(
            t16[:, :_DW] | (t16[:, _DW:] << 16), jnp.int32)


def _make_fused_table(embed, W, b):
    return pl.pallas_call(
        _table_body,
        out_shape=jax.ShapeDtypeStruct((_POS * _VOCAB, _DW), jnp.int32),
    )(embed, W, b)


def _sc_body(tok_ref, tbl_ref, out_ref,
             tok_v, idx_v, rows0, rows1, rows2, rows3,
             outc0, outc1, outc2, outc3,
             sem_g0, sem_g1, sem_g2, sem_g3,
             sem_o0, sem_o1, sem_o2, sem_o3,
             *, batch, b_per_w, cb):
    wid = lax.axis_index("s") * _NC + lax.axis_index("c")
    base = wid * b_per_w
    n_chunks = b_per_w // cb
    g = cb * _POS
    rows_bufs = (rows0, rows1, rows2, rows3)
    outc_bufs = (outc0, outc1, outc2, outc3)
    sem_g = (sem_g0, sem_g1, sem_g2, sem_g3)
    sem_o = (sem_o0, sem_o1, sem_o2, sem_o3)
    nbuf = 4
    sh16 = jnp.full((_LANES,), 16, jnp.int32)

    for j in range(_POS):
        pltpu.async_copy(tok_ref.at[pl.ds(j * batch + base, b_per_w)],
                         tok_v.at[pl.ds(j * b_per_w, b_per_w)], sem_o0)
    for j in range(_POS):
        pltpu.make_async_copy(tok_ref.at[pl.ds(0, b_per_w)],
                              tok_v.at[pl.ds(0, b_per_w)], sem_o0).wait()

    @plsc.parallel_loop(0, n_chunks, 1, unroll=2)
    def _(c):
        for j in range(_POS):
            for s in range(cb // _LANES):
                src = pl.ds(j * b_per_w + c * cb + s * _LANES, _LANES)
                idx_v[c, j // 2, pl.ds((j % 2) * cb + s * _LANES, _LANES)] = (
                    tok_v[src] + (j * _VOCAB))

    half = cb * _POS // 2

    def fire(c, rows_b, sem):
        pltpu.async_copy(tbl_ref.at[idx_v.at[c, 0]],
                         rows_b.at[pl.ds(0, half)], sem)
        pltpu.async_copy(tbl_ref.at[idx_v.at[c, 1]],
                         rows_b.at[pl.ds(half, half)], sem)

    for c0 in range(nbuf - 1):
        fire(c0, rows_bufs[c0], sem_g[c0])

    def pair_body(p, _):
        for bsel in range(nbuf):
            c = p * nbuf + bsel
            rows_b = rows_bufs[bsel]
            outc_b = outc_bufs[bsel]
            for h in range(2):
                pltpu.make_async_copy(tbl_ref.at[idx_v.at[0, 0]],
                                      rows_b.at[pl.ds(h * half, half)],
                                      sem_g[bsel]).wait()
            @pl.when(c >= nbuf)
            def _():
                pltpu.make_async_copy(outc_b, out_ref.at[pl.ds(0, cb)],
                                      sem_o[bsel]).wait()

            @plsc.parallel_loop(0, cb, 1, unroll=4)
            def _(r):
                for m in range(_DW // _LANES):
                    sl = pl.ds(m * _LANES, _LANES)
                    v = rows_b[r, sl]
                    acc_lo = plsc.bitcast(lax.shift_left(v, sh16), jnp.float32)
                    acc_hi = plsc.bitcast(v, jnp.float32)
                    for j in range(1, _POS):
                        v = rows_b[j * cb + r, sl]
                        acc_lo = acc_lo + plsc.bitcast(
                            lax.shift_left(v, sh16), jnp.float32)
                        acc_hi = acc_hi + plsc.bitcast(v, jnp.float32)
                    outc_b[r, pl.ds(m * 32, _LANES)] = acc_lo
                    outc_b[r, pl.ds(m * 32 + _LANES, _LANES)] = acc_hi

            row0 = base + c * cb
            pltpu.async_copy(outc_b, out_ref.at[pl.ds(row0, cb)],
                             sem_o[bsel])

            nsel = (bsel + nbuf - 1) % nbuf

            @pl.when(c + nbuf - 1 < n_chunks)
            def _():
                fire(c + nbuf - 1, rows_bufs[nsel], sem_g[nsel])
        return 0

    lax.fori_loop(0, n_chunks // nbuf, pair_body, 0)

    for bsel in range(nbuf):
        pltpu.make_async_copy(outc_bufs[bsel], out_ref.at[pl.ds(0, cb)],
                              sem_o[bsel]).wait()


def _gather_sum(tokens_pm, tbl, batch):
    b_per_w = batch // _NW
    cb = 32
    n_chunks = b_per_w // cb
    mesh = plsc.VectorSubcoreMesh(
        core_axis_name="c", subcore_axis_name="s",
        num_cores=_NC, num_subcores=_NS,
    )
    g = cb * _POS
    run = pl.kernel(
        functools.partial(_sc_body, batch=batch, b_per_w=b_per_w, cb=cb),
        out_type=jax.ShapeDtypeStruct((batch, _D), jnp.float32),
        mesh=mesh,
        compiler_params=pltpu.CompilerParams(
            needs_layout_passes=False,
            use_tc_tiling_on_sc=False,
        ),
        scratch_types=[
            pltpu.VMEM((_POS * b_per_w,), jnp.int32),
            pltpu.VMEM((n_chunks, 2, g // 2), jnp.int32),
            pltpu.VMEM((g, _DW), jnp.int32),
            pltpu.VMEM((g, _DW), jnp.int32),
            pltpu.VMEM((g, _DW), jnp.int32),
            pltpu.VMEM((g, _DW), jnp.int32),
            pltpu.VMEM((cb, _D), jnp.float32),
            pltpu.VMEM((cb, _D), jnp.float32),
            pltpu.VMEM((cb, _D), jnp.float32),
            pltpu.VMEM((cb, _D), jnp.float32),
            pltpu.SemaphoreType.DMA,
            pltpu.SemaphoreType.DMA,
            pltpu.SemaphoreType.DMA,
            pltpu.SemaphoreType.DMA,
            pltpu.SemaphoreType.DMA,
            pltpu.SemaphoreType.DMA,
            pltpu.SemaphoreType.DMA,
            pltpu.SemaphoreType.DMA,
        ],
    )
    return run(tokens_pm, tbl)


def kernel(ctrl_tokens, embed, W, b):
    batch = ctrl_tokens.shape[0]
    tokens_pm = ctrl_tokens.astype(jnp.int32).T.reshape(-1)
    tbl = _make_fused_table(embed, W, b)
    out = _gather_sum(tokens_pm, tbl, batch)
    return out[..., None]


# speedup vs baseline: 1.1363x; 1.1103x over previous
import functools

import jax
import jax.numpy as jnp
import numpy as np
from jax import lax
from jax.experimental import pallas as pl
from jax.experimental.pallas import tpu as pltpu
from jax.experimental.pallas import tpu_sc as plsc

_VOCAB = 1000
_D = 128
_DW = _D // 2
_POS = 4
_NC = 2
_NS = 16
_NW = _NC * _NS
_LANES = 16

def _table_body(embed_ref, w_ref, b_ref, tbl_ref):
    wp = jnp.concatenate(
        [w_ref[pl.ds(32 * m + h, 16), :] for h in (0, 16) for m in range(4)],
        axis=0)
    bp = jnp.concatenate(
        [b_ref[pl.ds(32 * m + h, 16)] for h in (0, 16) for m in range(4)])
    bias = bp * (1.0 / _POS)
    for j in range(_POS):
        wp_j = wp[:, j * _D:(j + 1) * _D]
        t = (
            lax.dot_general(
                embed_ref[...],
                wp_j,
                dimension_numbers=(((1,), (1,)), ((), ())),
                preferred_element_type=jnp.float32,
            )
            + bias[None, :]
        )
        t16 = lax.bitcast_convert_type(
            t.astype(jnp.bfloat16), jnp.uint16).astype(jnp.uint32)
        tbl_ref[pl.ds(j * _VOCAB, _VOCAB), :] = lax.bitcast_convert_type(
            t16[:, :_DW] | (t16[:, _DW:] << 16), jnp.int32)


def _make_fused_table(embed, W, b):
    return pl.pallas_call(
        _table_body,
        out_shape=jax.ShapeDtypeStruct((_POS * _VOCAB, _DW), jnp.int32),
    )(embed, W, b)


def _sc_body(tok_ref, tbl_ref, out_ref,
             tok_v, idx_v, rows0, rows1, rows2, rows3,
             outc0, outc1, outc2, outc3, tbl_sh,
             sem_g0, sem_g1, sem_g2, sem_g3,
             sem_o0, sem_o1, sem_o2, sem_o3,
             *, batch, b_per_w, cb):
    wid = lax.axis_index("s") * _NC + lax.axis_index("c")
    base = wid * b_per_w
    n_chunks = b_per_w // cb
    g = cb * _POS
    rows_bufs = (rows0, rows1, rows2, rows3)
    outc_bufs = (outc0, outc1, outc2, outc3)
    sem_g = (sem_g0, sem_g1, sem_g2, sem_g3)
    sem_o = (sem_o0, sem_o1, sem_o2, sem_o3)
    nbuf = 4
    sh16 = jnp.full((_LANES,), 16, jnp.int32)

    sid = lax.axis_index("s")
    stripe = (_POS * _VOCAB) // _NS
    pltpu.sync_copy(tbl_ref.at[pl.ds(sid * stripe, stripe)],
                    tbl_sh.at[pl.ds(sid * stripe, stripe)])

    for j in range(_POS):
        pltpu.async_copy(tok_ref.at[pl.ds(j * batch + base, b_per_w)],
                         tok_v.at[pl.ds(j * b_per_w, b_per_w)], sem_o0)
    for j in range(_POS):
        pltpu.make_async_copy(tok_ref.at[pl.ds(0, b_per_w)],
                              tok_v.at[pl.ds(0, b_per_w)], sem_o0).wait()

    @plsc.parallel_loop(0, n_chunks, 1, unroll=2)
    def _(c):
        for j in range(_POS):
            for s in range(cb // _LANES):
                src = pl.ds(j * b_per_w + c * cb + s * _LANES, _LANES)
                idx_v[c, j // 2, pl.ds((j % 2) * cb + s * _LANES, _LANES)] = (
                    tok_v[src] + (j * _VOCAB))

    half = cb * _POS // 2

    def fire(c, rows_b, sem):
        pltpu.async_copy(tbl_sh.at[idx_v.at[c, 0]],
                         rows_b.at[pl.ds(0, half)], sem)
        pltpu.async_copy(tbl_sh.at[idx_v.at[c, 1]],
                         rows_b.at[pl.ds(half, half)], sem)

    plsc.subcore_barrier()

    for c0 in range(nbuf - 1):
        fire(c0, rows_bufs[c0], sem_g[c0])

    def pair_body(p, _):
        for bsel in range(nbuf):
            c = p * nbuf + bsel
            rows_b = rows_bufs[bsel]
            outc_b = outc_bufs[bsel]
            for h in range(2):
                pltpu.make_async_copy(tbl_sh.at[idx_v.at[0, 0]],
                                      rows_b.at[pl.ds(h * half, half)],
                                      sem_g[bsel]).wait()
            @pl.when(c >= nbuf)
            def _():
                pltpu.make_async_copy(outc_b, out_ref.at[pl.ds(0, cb)],
                                      sem_o[bsel]).wait()

            @plsc.parallel_loop(0, cb, 1, unroll=4)
            def _(r):
                for m in range(_DW // _LANES):
                    sl = pl.ds(m * _LANES, _LANES)
                    v = rows_b[r, sl]
                    acc_lo = plsc.bitcast(lax.shift_left(v, sh16), jnp.float32)
                    acc_hi = plsc.bitcast(v, jnp.float32)
                    for j in range(1, _POS):
                        v = rows_b[j * cb + r, sl]
                        acc_lo = acc_lo + plsc.bitcast(
                            lax.shift_left(v, sh16), jnp.float32)
                        acc_hi = acc_hi + plsc.bitcast(v, jnp.float32)
                    outc_b[r, pl.ds(m * 32, _LANES)] = acc_lo
                    outc_b[r, pl.ds(m * 32 + _LANES, _LANES)] = acc_hi

            row0 = base + c * cb
            pltpu.async_copy(outc_b, out_ref.at[pl.ds(row0, cb)],
                             sem_o[bsel])

            nsel = (bsel + nbuf - 1) % nbuf

            @pl.when(c + nbuf - 1 < n_chunks)
            def _():
                fire(c + nbuf - 1, rows_bufs[nsel], sem_g[nsel])
        return 0

    lax.fori_loop(0, n_chunks // nbuf, pair_body, 0)

    for bsel in range(nbuf):
        pltpu.make_async_copy(outc_bufs[bsel], out_ref.at[pl.ds(0, cb)],
                              sem_o[bsel]).wait()


def _gather_sum(tokens_pm, tbl, batch):
    b_per_w = batch // _NW
    cb = 32
    n_chunks = b_per_w // cb
    mesh = plsc.VectorSubcoreMesh(
        core_axis_name="c", subcore_axis_name="s",
        num_cores=_NC, num_subcores=_NS,
    )
    g = cb * _POS
    run = pl.kernel(
        functools.partial(_sc_body, batch=batch, b_per_w=b_per_w, cb=cb),
        out_type=jax.ShapeDtypeStruct((batch, _D), jnp.float32),
        mesh=mesh,
        compiler_params=pltpu.CompilerParams(
            needs_layout_passes=False,
            use_tc_tiling_on_sc=False,
        ),
        scratch_types=[
            pltpu.VMEM((_POS * b_per_w,), jnp.int32),
            pltpu.VMEM((n_chunks, 2, g // 2), jnp.int32),
            pltpu.VMEM((g, _DW), jnp.int32),
            pltpu.VMEM((g, _DW), jnp.int32),
            pltpu.VMEM((g, _DW), jnp.int32),
            pltpu.VMEM((g, _DW), jnp.int32),
            pltpu.VMEM((cb, _D), jnp.float32),
            pltpu.VMEM((cb, _D), jnp.float32),
            pltpu.VMEM((cb, _D), jnp.float32),
            pltpu.VMEM((cb, _D), jnp.float32),
            pltpu.VMEM_SHARED((_POS * _VOCAB, _DW), jnp.int32),
            pltpu.SemaphoreType.DMA,
            pltpu.SemaphoreType.DMA,
            pltpu.SemaphoreType.DMA,
            pltpu.SemaphoreType.DMA,
            pltpu.SemaphoreType.DMA,
            pltpu.SemaphoreType.DMA,
            pltpu.SemaphoreType.DMA,
            pltpu.SemaphoreType.DMA,
        ],
    )
    return run(tokens_pm, tbl)


def kernel(ctrl_tokens, embed, W, b):
    batch = ctrl_tokens.shape[0]
    tokens_pm = ctrl_tokens.astype(jnp.int32).T.reshape(-1)
    tbl = _make_fused_table(embed, W, b)
    out = _gather_sum(tokens_pm, tbl, batch)
    return out[..., None]
